# baseline (speedup 1.0000x reference)
import functools

import jax
import jax.numpy as jnp
from jax import lax
from jax.experimental import pallas as pl
from jax.experimental.pallas import tpu as pltpu
from jax.experimental.pallas import tpu_sc as plsc

POS = 16
LANES = 16
NUM_CORES = 2
NUM_SUBCORES = 16
NW = NUM_CORES * NUM_SUBCORES


@functools.lru_cache(maxsize=None)
def _make_kernel(total, d_model):
    period = POS * d_model
    blk = 2 * period
    per_w = total // NW
    nblk = per_w // blk
    assert per_w % blk == 0 and total % NW == 0

    mesh = plsc.VectorSubcoreMesh(core_axis_name="c", subcore_axis_name="s")

    @functools.partial(
        pl.kernel,
        mesh=mesh,
        out_type=jax.ShapeDtypeStruct((total,), jnp.float32),
        scratch_types=[
            pltpu.VMEM((period,), jnp.float32),
            pltpu.VMEM((blk,), jnp.float32),
        ],
    )
    def run(x_hbm, t_hbm, o_hbm, tab_v, blk_v):
        wid = lax.axis_index("s") * NUM_CORES + lax.axis_index("c")
        base = wid * per_w
        pltpu.sync_copy(t_hbm, tab_v)

        def do_block(b, carry):
            off = base + b * blk
            pltpu.sync_copy(x_hbm.at[pl.ds(off, blk)], blk_v)

            def do_vec(j, c2):
                col = j * LANES
                tv = tab_v[pl.ds(col, LANES)]
                for t in range(blk // period):
                    p = t * period + col
                    blk_v[pl.ds(p, LANES)] = blk_v[pl.ds(p, LANES)] + tv
                return c2

            lax.fori_loop(0, period // LANES, do_vec, 0)
            pltpu.sync_copy(blk_v, o_hbm.at[pl.ds(off, blk)])
            return carry

        lax.fori_loop(0, nblk, do_block, 0)

    return run


def kernel(visn_feats, table):
    b, s, d = visn_feats.shape
    total = b * s * d
    x = visn_feats.reshape(total)
    t = table.reshape(POS * d)
    out = _make_kernel(total, d)(x, t)
    return out.reshape(b, s, d)


# speedup vs baseline: 2.0828x; 2.0828x over previous
import functools

import jax
import jax.numpy as jnp
from jax import lax
from jax.experimental import pallas as pl
from jax.experimental.pallas import tpu as pltpu
from jax.experimental.pallas import tpu_sc as plsc

POS = 16
LANES = 16
NUM_CORES = 2
NUM_SUBCORES = 16
NW = NUM_CORES * NUM_SUBCORES
NBUF = 4


@functools.lru_cache(maxsize=None)
def _make_kernel(total, d_model):
    period = POS * d_model
    blk = period
    per_w = total // NW
    nblk = per_w // blk
    assert total % NW == 0 and per_w % blk == 0 and nblk % NBUF == 0

    mesh = plsc.VectorSubcoreMesh(core_axis_name="c", subcore_axis_name="s")

    @functools.partial(
        pl.kernel,
        mesh=mesh,
        out_type=jax.ShapeDtypeStruct((total,), jnp.float32),
        scratch_types=[
            pltpu.VMEM((period,), jnp.float32),
            pltpu.VMEM((blk,), jnp.float32),
            pltpu.VMEM((blk,), jnp.float32),
            pltpu.VMEM((blk,), jnp.float32),
            pltpu.VMEM((blk,), jnp.float32),
            pltpu.SemaphoreType.DMA((NBUF,)),
            pltpu.SemaphoreType.DMA((NBUF,)),
        ],
    )
    def run(x_hbm, t_hbm, o_hbm, tab_v, b0, b1, b2, b3, in_sems, out_sems):
        bufs = (b0, b1, b2, b3)
        wid = lax.axis_index("s") * NUM_CORES + lax.axis_index("c")
        base = wid * per_w
        pltpu.sync_copy(t_hbm, tab_v)

        def in_copy(i, k):
            return pltpu.make_async_copy(
                x_hbm.at[pl.ds(base + i * blk, blk)], bufs[k], in_sems.at[k])

        def out_copy(i, k):
            return pltpu.make_async_copy(
                bufs[k], o_hbm.at[pl.ds(base + i * blk, blk)], out_sems.at[k])

        for k in range(NBUF - 1):
            in_copy(k, k).start()

        def do_slot(i, k):
            in_copy(i, k).wait()
            buf = bufs[k]

            @plsc.parallel_loop(0, blk, step=LANES, unroll=8)
            def _(p):
                buf[pl.ds(p, LANES)] = buf[pl.ds(p, LANES)] + tab_v[pl.ds(p, LANES)]

            out_copy(i, k).start()
            nxt = i + NBUF - 1
            kn = (k + NBUF - 1) % NBUF

            @pl.when(nxt < nblk)
            def _():
                @pl.when(nxt >= NBUF)
                def _():
                    out_copy(nxt - NBUF, kn).wait()
                in_copy(nxt, kn).start()

        def do_group(q, carry):
            b = q * NBUF
            for k in range(NBUF):
                do_slot(b + k, k)
            return carry

        lax.fori_loop(0, nblk // NBUF, do_group, 0)
        for k in range(NBUF):
            out_copy(nblk - NBUF + k, k).wait()

    return run


def kernel(visn_feats, table):
    b, s, d = visn_feats.shape
    total = b * s * d
    x = visn_feats.reshape(total)
    t = table.reshape(POS * d)
    out = _make_kernel(total, d)(x, t)
    return out.reshape(b, s, d)
